# Initial kernel scaffold; baseline (speedup 1.0000x reference)
import jax
import jax.numpy as jnp
from jax.experimental import pallas as pl


def kernel(x, latent_tdirs):
    raise NotImplementedError("write your pallas kernel here")


# speedup vs baseline: 4.8538x; 4.8538x over previous
import functools

import jax
import jax.numpy as jnp
from jax import lax
from jax.experimental import pallas as pl
from jax.experimental.pallas import tpu as pltpu
from jax.experimental.pallas import tpu_sc as plsc

DICT_LEN = 1000000
LATENT_SIZE = 32

G = 128
K = 16

NC = 2
NS = 16
NW = NC * NS


def _embed_kernel(n_iters, idx_hbm, table_hbm, out_hbm, idx_v, rows_v, sem):
    wid = lax.axis_index("s") * NC + lax.axis_index("c")
    groups_per_w = n_iters * K
    base = wid * groups_per_w

    def body(g, carry):
        g0 = base + g * K
        pltpu.sync_copy(idx_hbm.at[pl.ds(g0, K)], idx_v)
        copies = []
        for j in range(K):
            copies.append(
                pltpu.async_copy(table_hbm.at[idx_v.at[j]], rows_v.at[j], sem)
            )
        for c in copies:
            c.wait()
        pltpu.sync_copy(rows_v, out_hbm.at[pl.ds(g0, K)])
        return carry

    lax.fori_loop(0, n_iters, body, 0)


def kernel(x, latent_tdirs):
    orig_shape = x.shape[:-1] + (LATENT_SIZE,)
    idx = jnp.reshape(x, (-1,)).astype(jnp.int32)
    n = idx.shape[0]
    assert n % (NW * K * G) == 0
    n_iters = n // (NW * K * G)

    idx_g = jnp.reshape(idx, (n // G, G))

    mesh = plsc.VectorSubcoreMesh(core_axis_name="c", subcore_axis_name="s")
    run = functools.partial(
        pl.kernel,
        mesh=mesh,
        compiler_params=pltpu.CompilerParams(use_tc_tiling_on_sc=False),
        out_type=jax.ShapeDtypeStruct((n // G, G, LATENT_SIZE), jnp.float32),
        scratch_types=[
            pltpu.VMEM((K, G), jnp.int32),
            pltpu.VMEM((K, G, LATENT_SIZE), jnp.float32),
            pltpu.SemaphoreType.DMA,
        ],
    )(functools.partial(_embed_kernel, n_iters))

    out = run(idx_g, latent_tdirs)
    return jnp.reshape(out, orig_shape)


# speedup vs baseline: 4.8819x; 1.0058x over previous
import functools

import jax
import jax.numpy as jnp
from jax import lax
from jax.experimental import pallas as pl
from jax.experimental.pallas import tpu as pltpu
from jax.experimental.pallas import tpu_sc as plsc

DICT_LEN = 1000000
LATENT_SIZE = 32

G = 128
K = 10

NC = 2
NS = 16
NW = NC * NS


def _embed_kernel(n_iters, idx_hbm, table_hbm, out_hbm, idx_v, rows_v,
                  sem0, sem1):
    wid = lax.axis_index("s") * NC + lax.axis_index("c")
    base = wid * n_iters * K
    sems = (sem0, sem1)

    def stage_and_fire(s, g):
        pltpu.sync_copy(idx_hbm.at[pl.ds(base + g * K, K)], idx_v.at[s])
        for j in range(K):
            pltpu.async_copy(
                table_hbm.at[idx_v.at[s].at[j]], rows_v.at[s].at[j], sems[s]
            )

    def drain_and_write(s, g):
        pltpu.make_async_copy(
            out_hbm.at[pl.ds(0, K)], rows_v.at[s], sems[s]
        ).wait()
        pltpu.sync_copy(rows_v.at[s], out_hbm.at[pl.ds(base + g * K, K)])

    stage_and_fire(0, 0)

    def body(p, carry):
        g = 2 * p
        stage_and_fire(1, g + 1)
        drain_and_write(0, g)
        stage_and_fire(0, g + 2)
        drain_and_write(1, g + 1)
        return carry

    lax.fori_loop(0, n_iters // 2 - 1, body, 0)

    g = n_iters - 2
    stage_and_fire(1, g + 1)
    drain_and_write(0, g)
    drain_and_write(1, g + 1)


def kernel(x, latent_tdirs):
    orig_shape = x.shape[:-1] + (LATENT_SIZE,)
    idx = jnp.reshape(x, (-1,)).astype(jnp.int32)
    n = idx.shape[0]
    assert n % (NW * K * G) == 0
    n_iters = n // (NW * K * G)

    idx_g = jnp.reshape(idx, (n // G, G))

    mesh = plsc.VectorSubcoreMesh(core_axis_name="c", subcore_axis_name="s")
    run = functools.partial(
        pl.kernel,
        mesh=mesh,
        compiler_params=pltpu.CompilerParams(use_tc_tiling_on_sc=False),
        out_type=jax.ShapeDtypeStruct((n // G, G, LATENT_SIZE), jnp.float32),
        scratch_types=[
            pltpu.VMEM((2, K, G), jnp.int32),
            pltpu.VMEM((2, K, G, LATENT_SIZE), jnp.float32),
            pltpu.SemaphoreType.DMA,
            pltpu.SemaphoreType.DMA,
        ],
    )(functools.partial(_embed_kernel, n_iters))

    out = run(idx_g, latent_tdirs)
    return jnp.reshape(out, orig_shape)
